# baseline (speedup 1.0000x reference)
import jax
import jax.numpy as jnp
from jax.experimental import pallas as pl
from jax.experimental.pallas import tpu as pltpu

NUM_TOKENS = 8192
D_MODEL = 2048
NUM_EXPERTS = 16
BLOCK_T = 1024


def _gating_kernel(x_ref, w_ref, b_ref, s_ref, ones_ref, eye_ref,
                   weights_ref, logits_ref):
    acc = jax.lax.dot_general(
        w_ref[...], x_ref[...],
        dimension_numbers=(((1,), (1,)), ((), ())),
        preferred_element_type=jnp.float32,
    )
    acc = acc + b_ref[...]
    clean = acc[:NUM_EXPERTS, :]
    raw_noise = acc[NUM_EXPERTS:, :]
    noise_std = jnp.log1p(jnp.exp(raw_noise))
    s_t = jax.lax.dot_general(
        eye_ref[...], s_ref[...],
        dimension_numbers=(((1,), (1,)), ((), ())),
        preferred_element_type=jnp.float32,
    )
    logits = clean + s_t * noise_std
    e = jnp.exp(logits)
    s = jnp.dot(ones_ref[...], e, preferred_element_type=jnp.float32)
    weights = e / s
    weights_ref[...] = jax.lax.dot_general(
        weights, eye_ref[...],
        dimension_numbers=(((0,), (0,)), ((), ())),
        preferred_element_type=jnp.float32,
    )
    logits_ref[...] = jax.lax.dot_general(
        logits, eye_ref[...],
        dimension_numbers=(((0,), (0,)), ((), ())),
        preferred_element_type=jnp.float32,
    )


def kernel(x, Wg, bg, Wn, bn):
    T, D = x.shape
    E = Wg.shape[0]
    w = jnp.concatenate([Wg, Wn], axis=0)
    b = jnp.concatenate([bg, bn], axis=0)[:, None]
    sample = jax.random.normal(jax.random.key(42), (T, E), dtype=x.dtype)
    ones = jnp.ones((E, E), dtype=x.dtype)
    eye = jnp.eye(E, dtype=x.dtype)

    grid = (T // BLOCK_T,)
    out_shape = [
        jax.ShapeDtypeStruct((T, E), x.dtype),
        jax.ShapeDtypeStruct((T, E), x.dtype),
    ]
    weights, logits = pl.pallas_call(
        _gating_kernel,
        grid=grid,
        in_specs=[
            pl.BlockSpec((BLOCK_T, D), lambda i: (i, 0)),
            pl.BlockSpec((2 * E, D), lambda i: (0, 0)),
            pl.BlockSpec((2 * E, 1), lambda i: (0, 0)),
            pl.BlockSpec((BLOCK_T, E), lambda i: (i, 0)),
            pl.BlockSpec((E, E), lambda i: (0, 0)),
            pl.BlockSpec((E, E), lambda i: (0, 0)),
        ],
        out_specs=[
            pl.BlockSpec((BLOCK_T, E), lambda i: (i, 0)),
            pl.BlockSpec((BLOCK_T, E), lambda i: (i, 0)),
        ],
        out_shape=out_shape,
        compiler_params=pltpu.CompilerParams(
            dimension_semantics=("arbitrary",),
        ),
    )(x, w, b, sample, ones, eye)
    return (weights, logits)


# speedup vs baseline: 1.8764x; 1.8764x over previous
import jax
import jax.numpy as jnp
from jax.experimental import pallas as pl
from jax.experimental.pallas import tpu as pltpu

NUM_TOKENS = 8192
D_MODEL = 2048
NUM_EXPERTS = 16
BLOCK_T = 512
NBUF = 6
LOOKAHEAD = 4


def _copy_block(x_hbm, xbuf, sems, k):
    slot = jax.lax.rem(k, NBUF)
    return pltpu.make_async_copy(
        x_hbm.at[pl.ds(k * BLOCK_T, BLOCK_T), :],
        xbuf.at[slot],
        sems.at[slot],
    )


def _gating_kernel(x_hbm, w_ref, b_ref, s_ref, ones_ref,
                   weights_ref, logits_ref, xbuf, sems):
    i = pl.program_id(0)
    n = pl.num_programs(0)

    @pl.when(i == 0)
    def _prologue():
        for k in range(LOOKAHEAD + 1):
            _copy_block(x_hbm, xbuf, sems, k).start()

    @pl.when(i + LOOKAHEAD + 1 < n)
    def _issue_next():
        _copy_block(x_hbm, xbuf, sems, i + LOOKAHEAD + 1).start()

    _copy_block(x_hbm, xbuf, sems, i).wait()
    xblk = xbuf[jax.lax.rem(i, NBUF)]

    acc = jax.lax.dot_general(
        w_ref[...], xblk,
        dimension_numbers=(((1,), (1,)), ((), ())),
        preferred_element_type=jnp.float32,
    )
    acc = acc + b_ref[...]
    clean = acc[:NUM_EXPERTS, :]
    raw_noise = acc[NUM_EXPERTS:, :]
    noise_std = jnp.log1p(jnp.exp(raw_noise))
    logits = clean + s_ref[...] * noise_std
    e = jnp.exp(logits)
    s = jnp.dot(ones_ref[...], e, preferred_element_type=jnp.float32)
    weights_ref[...] = e / s
    logits_ref[...] = logits


def kernel(x, Wg, bg, Wn, bn):
    T, D = x.shape
    E = Wg.shape[0]
    w = jnp.concatenate([Wg, Wn], axis=0)
    b = jnp.concatenate([bg, bn], axis=0)[:, None]
    sample_t = jax.random.normal(jax.random.key(42), (T, E), dtype=x.dtype).T
    ones = jnp.ones((E, E), dtype=x.dtype)

    grid = (T // BLOCK_T,)
    out_shape = [
        jax.ShapeDtypeStruct((E, T), x.dtype),
        jax.ShapeDtypeStruct((E, T), x.dtype),
    ]
    weights_t, logits_t = pl.pallas_call(
        _gating_kernel,
        grid=grid,
        in_specs=[
            pl.BlockSpec(memory_space=pltpu.MemorySpace.HBM),
            pl.BlockSpec((2 * E, D), lambda i: (0, 0)),
            pl.BlockSpec((2 * E, 1), lambda i: (0, 0)),
            pl.BlockSpec((E, BLOCK_T), lambda i: (0, i)),
            pl.BlockSpec((E, E), lambda i: (0, 0)),
        ],
        out_specs=[
            pl.BlockSpec((E, BLOCK_T), lambda i: (0, i)),
            pl.BlockSpec((E, BLOCK_T), lambda i: (0, i)),
        ],
        out_shape=out_shape,
        scratch_shapes=[
            pltpu.VMEM((NBUF, BLOCK_T, D), jnp.float32),
            pltpu.SemaphoreType.DMA((NBUF,)),
        ],
        compiler_params=pltpu.CompilerParams(
            dimension_semantics=("arbitrary",),
        ),
    )(x, w, b, sample_t, ones)
    return (weights_t.T, logits_t.T)
